# Initial kernel scaffold; baseline (speedup 1.0000x reference)
import jax
import jax.numpy as jnp
from jax.experimental import pallas as pl


def kernel(x, params, edge_index, batch):
    raise NotImplementedError("write your pallas kernel here")


# speedup vs baseline: 12.9540x; 12.9540x over previous
import functools

import jax
import jax.numpy as jnp
from jax import lax
from jax.experimental import pallas as pl
from jax.experimental.pallas import tpu as pltpu
from jax.experimental.pallas import tpu_sc as plsc

N = 10000
E = 320000
D = 128
H = 128
G = 64
S = 64

NC = 2
NS = 16
NW = NC * NS

E_CHUNK = 64
E_BLOCKS = 6
E_CHUNKS_PER_B = 27
E_CHUNKS_PER_W = E_BLOCKS * E_CHUNKS_PER_B
EP = NW * E_CHUNKS_PER_W * E_CHUNK
E_PAD_ROW = N

TBL = 10240
TBL_PER_TILE = TBL // NS

NPT = NW * 3 * 128

P_CHUNK = 64
P_CHUNKS_PER_W = 5

SEG_R = 80
SEG_TRASH = 64

_MESH = plsc.VectorSubcoreMesh(
    core_axis_name="c", subcore_axis_name="s", num_cores=NC, num_subcores=NS
)


def _fill_f32(ref, rows, value):
  cols = ref.shape[1]
  vec = jnp.full((16,), value, dtype=jnp.float32)
  per_row = cols // 16

  def body(i, _):
    r = i // per_row
    c = (i % per_row) * 16
    ref[r, pl.ds(c, 16)] = vec
    return 0

  lax.fori_loop(0, rows * per_row, body, 0)


@functools.partial(
    pl.kernel,
    mesh=_MESH,
    out_type=jax.ShapeDtypeStruct((NC, TBL, H), jnp.float32),
    scratch_types=[
        pltpu.VMEM_SHARED((TBL, H), jnp.float32),
        pltpu.VMEM((E_CHUNKS_PER_B, E_CHUNK), jnp.int32),
        pltpu.VMEM((E_CHUNKS_PER_B, E_CHUNK), jnp.int32),
        pltpu.VMEM((E_CHUNK, H), jnp.float32),
        pltpu.VMEM((E_CHUNK, H), jnp.float32),
        pltpu.VMEM((32, H), jnp.float32),
        pltpu.SemaphoreType.DMA,
        pltpu.SemaphoreType.DMA,
    ],
)
def _sc_message(table, src_hbm, dst_hbm, out, acc, src_v, dst_v, rows0, rows1,
                zbuf, gsem0, gsem1):
  c = lax.axis_index("c")
  s = lax.axis_index("s")
  w = s * NC + c

  _fill_f32(zbuf, 32, 0.0)
  for k in range(TBL_PER_TILE // 32):
    pltpu.sync_copy(zbuf, acc.at[pl.ds(s * TBL_PER_TILE + k * 32, 32)])

  plsc.subcore_barrier()

  for blk in range(E_BLOCKS):
    pltpu.sync_copy(src_hbm.at[w, blk], src_v)
    pltpu.sync_copy(dst_hbm.at[w, blk], dst_v)

    cp0 = pltpu.async_copy(table.at[src_v.at[0]], rows0, gsem0)
    pltpu.async_copy(table.at[src_v.at[1]], rows1, gsem1)
    cp0.wait()

    def body(j, _):
      @pl.when(j % 2 == 0)
      def _():
        pltpu.sync_copy(rows0, acc.at[dst_v.at[j]], add=True)

        @pl.when(j + 2 < E_CHUNKS_PER_B)
        def _():
          pltpu.async_copy(table.at[src_v.at[j + 2]], rows0, gsem0)

        @pl.when(j + 1 < E_CHUNKS_PER_B)
        def _():
          pltpu.make_async_copy(table.at[src_v.at[j + 1]], rows1, gsem1).wait()

      @pl.when(j % 2 == 1)
      def _():
        pltpu.sync_copy(rows1, acc.at[dst_v.at[j]], add=True)

        @pl.when(j + 2 < E_CHUNKS_PER_B)
        def _():
          pltpu.async_copy(table.at[src_v.at[j + 2]], rows1, gsem1)

        @pl.when(j + 1 < E_CHUNKS_PER_B)
        def _():
          pltpu.make_async_copy(table.at[src_v.at[j + 1]], rows0, gsem0).wait()

      return 0

    lax.fori_loop(0, E_CHUNKS_PER_B, body, 0)

  plsc.subcore_barrier()

  sl = pl.ds(s * TBL_PER_TILE, TBL_PER_TILE)
  pltpu.sync_copy(acc.at[sl], out.at[c, sl])


@functools.partial(
    pl.kernel,
    mesh=_MESH,
    out_type=(
        jax.ShapeDtypeStruct((NC, TBL, H), jnp.float32),
        jax.ShapeDtypeStruct((NC, SEG_R, D), jnp.float32),
        jax.ShapeDtypeStruct((NC, SEG_R, H), jnp.float32),
    ),
    scratch_types=[
        pltpu.VMEM_SHARED((TBL, H), jnp.float32),
        pltpu.VMEM_SHARED((SEG_R, D), jnp.float32),
        pltpu.VMEM_SHARED((SEG_R, H), jnp.float32),
        pltpu.VMEM((E_CHUNKS_PER_B, E_CHUNK), jnp.int32),
        pltpu.VMEM((3, 128), jnp.int32),
        pltpu.VMEM((128, D), jnp.float32),
        pltpu.VMEM((128, H), jnp.float32),
        pltpu.SemaphoreType.DMA,
    ],
)
def _sc_stats(x_hbm, sid_hbm, dst_hbm, out_deg, out_sum, out_cnt,
              deg_acc, sum_acc, cnt_acc, dst_v, sid_v, xrows, ones, gsem):
  c = lax.axis_index("c")
  s = lax.axis_index("s")
  w = s * NC + c

  _fill_f32(ones, 128, 0.0)
  for k in range(TBL_PER_TILE // 128):
    pltpu.sync_copy(ones, deg_acc.at[pl.ds(s * TBL_PER_TILE + k * 128, 128)])

  @pl.when(s == 0)
  def _():
    pltpu.sync_copy(ones.at[pl.ds(0, SEG_R)], cnt_acc)

  @pl.when(s == 1)
  def _():
    pltpu.sync_copy(ones.at[pl.ds(0, SEG_R)], sum_acc)

  _fill_f32(ones, 128, 1.0)
  pltpu.sync_copy(sid_hbm.at[w], sid_v)

  plsc.subcore_barrier()

  def dbody(j, _):
    pltpu.sync_copy(ones.at[pl.ds(0, E_CHUNK)], deg_acc.at[dst_v.at[j]],
                    add=True)
    return 0

  for blk in range(E_BLOCKS):
    pltpu.sync_copy(dst_hbm.at[w, blk], dst_v)
    lax.fori_loop(0, E_CHUNKS_PER_B, dbody, 0)

  for j in range(3):
    pltpu.async_copy(
        x_hbm.at[pl.ds(w * 384 + j * 128, 128)], xrows, gsem).wait()
    pltpu.sync_copy(xrows, sum_acc.at[sid_v.at[j]], add=True)
    pltpu.sync_copy(ones, cnt_acc.at[sid_v.at[j]], add=True)

  plsc.subcore_barrier()

  sl = pl.ds(s * TBL_PER_TILE, TBL_PER_TILE)
  pltpu.sync_copy(deg_acc.at[sl], out_deg.at[c, sl])

  @pl.when(s == 2)
  def _():
    pltpu.sync_copy(sum_acc, out_sum.at[c])

  @pl.when(s == 3)
  def _():
    pltpu.sync_copy(cnt_acc, out_cnt.at[c])


@functools.partial(
    pl.kernel,
    mesh=_MESH,
    out_type=(
        jax.ShapeDtypeStruct((NC, SEG_R, H), jnp.float32),
        jax.ShapeDtypeStruct((NC, SEG_R, H), jnp.float32),
    ),
    scratch_types=[
        pltpu.VMEM_SHARED((SEG_R, H), jnp.float32),
        pltpu.VMEM_SHARED((SEG_R, H), jnp.float32),
        pltpu.VMEM((P_CHUNKS_PER_W, P_CHUNK), jnp.int32),
        pltpu.VMEM((P_CHUNK, H), jnp.float32),
        pltpu.VMEM((P_CHUNK, H), jnp.float32),
        pltpu.SemaphoreType.DMA,
    ],
)
def _sc_pool(h_hbm, bat_hbm, out_sum, out_cnt, sum_acc, cnt_acc, bat_v, hrows,
             ones, gsem):
  c = lax.axis_index("c")
  s = lax.axis_index("s")
  w = s * NC + c

  _fill_f32(ones, P_CHUNK, 0.0)

  @pl.when(s == 0)
  def _():
    pltpu.sync_copy(ones, sum_acc.at[pl.ds(0, 64)])
    pltpu.sync_copy(ones, sum_acc.at[pl.ds(SEG_R - 64, 64)])

  @pl.when(s == 1)
  def _():
    pltpu.sync_copy(ones, cnt_acc.at[pl.ds(0, 64)])
    pltpu.sync_copy(ones, cnt_acc.at[pl.ds(SEG_R - 64, 64)])

  _fill_f32(ones, P_CHUNK, 1.0)
  pltpu.sync_copy(bat_hbm.at[w], bat_v)

  plsc.subcore_barrier()

  for j in range(P_CHUNKS_PER_W):
    pltpu.async_copy(
        h_hbm.at[pl.ds(w * P_CHUNKS_PER_W * P_CHUNK + j * P_CHUNK, P_CHUNK)],
        hrows, gsem).wait()
    pltpu.sync_copy(hrows, sum_acc.at[bat_v.at[j]], add=True)
    pltpu.sync_copy(ones, cnt_acc.at[bat_v.at[j]], add=True)

  plsc.subcore_barrier()

  @pl.when(s == 0)
  def _():
    pltpu.sync_copy(sum_acc, out_sum.at[c])

  @pl.when(s == 1)
  def _():
    pltpu.sync_copy(cnt_acc, out_cnt.at[c])


def _elu(x):
  return jnp.where(x > 0, x, jnp.exp(x) - 1.0)


def _mm(a, b):
  return jax.lax.dot_general(a, b, (((1,), (0,)), ((), ())),
                             preferred_element_type=jnp.float32)


def _tc_dinv_body(deg_ref, out_ref):
  deg = deg_ref[0, :, 0:1] + deg_ref[1, :, 0:1]
  out_ref[...] = jnp.where(deg > 0, jax.lax.rsqrt(deg), 0.0)


def _tc_dinv(deg_p):
  blk = 1024
  return pl.pallas_call(
      _tc_dinv_body,
      grid=(TBL // blk,),
      in_specs=[pl.BlockSpec((NC, blk, H), lambda b: (0, b, 0))],
      out_specs=pl.BlockSpec((blk, 1), lambda b: (b, 0)),
      out_shape=jax.ShapeDtypeStruct((TBL, 1), jnp.float32),
  )(deg_p)


def _tc_attn_body(sum_ref, cnt_ref, w1_ref, b1_ref, w2_ref, out_ref):
  cnt = cnt_ref[0, :S, 0:1] + cnt_ref[1, :S, 0:1]
  sums = sum_ref[0, :S, :] + sum_ref[1, :S, :]
  sm = sums / jnp.maximum(cnt, 1.0)
  hdn = jnp.tanh(_mm(sm, w1_ref[...]) + b1_ref[...])
  scores = _mm(hdn, w2_ref[...])
  scores = jnp.where(cnt > 0, scores, -1e30)
  m = jnp.max(scores)
  e = jnp.exp(scores - m)
  out_ref[...] = e / jnp.sum(e)


def _tc_attn(sums_p, cnt_p, w1, b1, w2):
  return pl.pallas_call(
      _tc_attn_body,
      out_shape=jax.ShapeDtypeStruct((S, 1), jnp.float32),
  )(sums_p, cnt_p, w1, b1, w2)


def _tc_prep_body(x_ref, dinv_ref, attn_ref, w0a_ref, w0b_ref, out_ref):
  x = x_ref[...]
  sid = x[:, 5:6].astype(jnp.int32)
  onehot = (sid == lax.broadcasted_iota(jnp.int32, (1, S), 1)
            ).astype(jnp.float32)
  attn = _mm(onehot, attn_ref[...])
  hw = _mm(x, w0a_ref[...]) + attn * w0b_ref[...]
  out_ref[...] = hw * dinv_ref[...]


def _tc_prep(x_pad, dinv, attn, w0a, w0b):
  blk = 1024
  return pl.pallas_call(
      _tc_prep_body,
      grid=(TBL // blk,),
      in_specs=[
          pl.BlockSpec((blk, D), lambda b: (b, 0)),
          pl.BlockSpec((blk, 1), lambda b: (b, 0)),
          pl.BlockSpec((S, 1), lambda b: (0, 0)),
          pl.BlockSpec((D, H), lambda b: (0, 0)),
          pl.BlockSpec((1, H), lambda b: (0, 0)),
      ],
      out_specs=pl.BlockSpec((blk, H), lambda b: (b, 0)),
      out_shape=jax.ShapeDtypeStruct((TBL, H), jnp.float32),
  )(x_pad, dinv, attn, w0a, w0b)


def _tc_combine_body(p_ref, dinv_ref, b_ref, t_ref, st_ref):
  blk = t_ref.shape[0]
  b = pl.program_id(0)
  t = (p_ref[0] + p_ref[1]) * dinv_ref[...] + b_ref[...]
  t_ref[...] = t
  rows = b * blk + lax.broadcasted_iota(jnp.int32, (blk, 1), 0)
  tm = jnp.where(rows < N, t, 0.0)

  @pl.when(b == 0)
  def _():
    st_ref[...] = jnp.zeros_like(st_ref)

  st_ref[0:1, :] += jnp.sum(tm, axis=0, keepdims=True)
  st_ref[1:2, :] += jnp.sum(tm * tm, axis=0, keepdims=True)


def _tc_combine(p, dinv, bias):
  blk = 1024
  return pl.pallas_call(
      _tc_combine_body,
      grid=(TBL // blk,),
      in_specs=[
          pl.BlockSpec((NC, blk, H), lambda b: (0, b, 0)),
          pl.BlockSpec((blk, 1), lambda b: (b, 0)),
          pl.BlockSpec((1, H), lambda b: (0, 0)),
      ],
      out_specs=[
          pl.BlockSpec((blk, H), lambda b: (b, 0)),
          pl.BlockSpec((8, H), lambda b: (0, 0)),
      ],
      out_shape=[
          jax.ShapeDtypeStruct((TBL, H), jnp.float32),
          jax.ShapeDtypeStruct((8, H), jnp.float32),
      ],
  )(p, dinv, bias)


def _bn_act(t_ref, st_ref, g_ref, bb_ref, last):
  mu = st_ref[0:1, :] * (1.0 / N)
  ex2 = st_ref[1:2, :] * (1.0 / N)
  var = ex2 - mu * mu
  inv = jax.lax.rsqrt(var + 1e-5)
  hn = (t_ref[...] - mu) * (inv * g_ref[...]) + bb_ref[...]
  return jnp.maximum(hn, 0.0) if last else _elu(hn)


def _tc_norm_mm_body(t_ref, st_ref, g_ref, bb_ref, w_ref, dinv_ref, out_ref):
  h = _bn_act(t_ref, st_ref, g_ref, bb_ref, last=False)
  out_ref[...] = _mm(h, w_ref[...]) * dinv_ref[...]


def _tc_norm_mm(t, st, g, bb, w, dinv):
  blk = 1024
  return pl.pallas_call(
      _tc_norm_mm_body,
      grid=(TBL // blk,),
      in_specs=[
          pl.BlockSpec((blk, H), lambda b: (b, 0)),
          pl.BlockSpec((8, H), lambda b: (0, 0)),
          pl.BlockSpec((1, H), lambda b: (0, 0)),
          pl.BlockSpec((1, H), lambda b: (0, 0)),
          pl.BlockSpec((H, H), lambda b: (0, 0)),
          pl.BlockSpec((blk, 1), lambda b: (b, 0)),
      ],
      out_specs=pl.BlockSpec((blk, H), lambda b: (b, 0)),
      out_shape=jax.ShapeDtypeStruct((TBL, H), jnp.float32),
  )(t, st, g, bb, w, dinv)


def _tc_norm_last_body(t_ref, st_ref, g_ref, bb_ref, out_ref):
  out_ref[...] = _bn_act(t_ref, st_ref, g_ref, bb_ref, last=True)


def _tc_norm_last(t, st, g, bb):
  blk = 1024
  return pl.pallas_call(
      _tc_norm_last_body,
      grid=(TBL // blk,),
      in_specs=[
          pl.BlockSpec((blk, H), lambda b: (b, 0)),
          pl.BlockSpec((8, H), lambda b: (0, 0)),
          pl.BlockSpec((1, H), lambda b: (0, 0)),
          pl.BlockSpec((1, H), lambda b: (0, 0)),
      ],
      out_specs=pl.BlockSpec((blk, H), lambda b: (b, 0)),
      out_shape=jax.ShapeDtypeStruct((TBL, H), jnp.float32),
  )(t, st, g, bb)


def _tc_head_body(gs_ref, gc_ref, w1_ref, b1_ref, w2_ref, b2_ref, out_ref):
  cnt = gc_ref[0, :G, 0:1] + gc_ref[1, :G, 0:1]
  gsum = gs_ref[0, :G, :] + gs_ref[1, :G, :]
  g = gsum / jnp.maximum(cnt, 1.0)
  z = _elu(_mm(g, w1_ref[...]) + b1_ref[...])
  out_ref[...] = _mm(z, w2_ref[...]) + b2_ref[...]


def _tc_head(gs_p, gc_p, w1, b1, w2, b2):
  return pl.pallas_call(
      _tc_head_body,
      out_shape=jax.ShapeDtypeStruct((G, 1), jnp.float32),
  )(gs_p, gc_p, w1, b1, w2, b2)


_USE_SC_STATS = True
_USE_SC_MSG = True
_USE_SC_POOL = True
_USE_TC = True


def kernel(x, params, edge_index, batch):
  sid = x[:, 5].astype(jnp.int32)
  sid_pad = jnp.full((NPT,), SEG_TRASH, jnp.int32).at[:N].set(sid)
  sid_idx = sid_pad.reshape(NW, 3, 128)

  x_pad = jnp.zeros((NPT, D), jnp.float32).at[:N].set(x)

  loop = jnp.arange(N, dtype=jnp.int32)
  epad = jnp.full((EP - E - N,), E_PAD_ROW, jnp.int32)
  src = jnp.concatenate([edge_index[0].astype(jnp.int32), loop, epad])
  dst = jnp.concatenate([edge_index[1].astype(jnp.int32), loop, epad])
  src_idx = src.reshape(NW, E_BLOCKS, E_CHUNKS_PER_B, E_CHUNK)
  dst_idx = dst.reshape(NW, E_BLOCKS, E_CHUNKS_PER_B, E_CHUNK)

  bat_pad = jnp.full((TBL,), SEG_TRASH, jnp.int32).at[:N].set(
      batch.astype(jnp.int32))
  bat_idx = bat_pad.reshape(NW, P_CHUNKS_PER_W, P_CHUNK)

  p = params
  r1 = lambda a: a.reshape(1, -1)
  half = jnp.array([0.5, 0.5]).reshape(NC, 1, 1)

  if _USE_SC_STATS:
    deg_p, sum_p, cnt_p = _sc_stats(x_pad, sid_idx, dst_idx)
  else:
    deg_p = jax.ops.segment_sum(
        jnp.ones((EP, H), jnp.float32), dst, num_segments=TBL
    ).reshape(1, TBL, H) * half
    sum_p = jax.ops.segment_sum(x, sid, num_segments=SEG_R).reshape(
        1, SEG_R, D) * half
    cnt_p = jax.ops.segment_sum(
        jnp.ones((N, H), jnp.float32), sid, num_segments=SEG_R
    ).reshape(1, SEG_R, H) * half

  if _USE_TC:
    dinv = _tc_dinv(deg_p)
    attn = _tc_attn(sum_p, cnt_p, p['attn_w1'], r1(p['attn_b1']),
                    p['attn_w2'])
    w0 = p['conv_w0']
    hws = _tc_prep(x_pad[:TBL], dinv, attn, w0[:D], w0[D:D + 1])
  else:
    cnt = (cnt_p[0] + cnt_p[1])[:S, 0:1]
    sums = (sum_p[0] + sum_p[1])[:S]
    sm = sums / jnp.maximum(cnt, 1.0)
    hdn = jnp.tanh(sm @ p['attn_w1'] + p['attn_b1'])
    scores = jnp.where(cnt > 0, hdn @ p['attn_w2'], -1e30)
    e = jnp.exp(scores - jnp.max(scores))
    attn = e / jnp.sum(e)
    deg = (deg_p[0] + deg_p[1])[:, 0:1]
    dinv = jnp.where(deg > 0, jax.lax.rsqrt(deg), 0.0)
    sid_t = jnp.zeros((TBL,), jnp.int32).at[:N].set(sid)
    av = attn[sid_t]
    w0 = p['conv_w0']
    hws = (x_pad[:TBL] @ w0[:D] + av * w0[D:D + 1]) * dinv

  for i in range(4):
    if _USE_SC_MSG:
      part = _sc_message(hws, src_idx, dst_idx)
    else:
      part = jax.ops.segment_sum(hws[src], dst, num_segments=TBL).reshape(
          1, TBL, H) * half
    if _USE_TC:
      t, st = _tc_combine(part, dinv, r1(p['conv_b%d' % i]))
      if i < 3:
        hws = _tc_norm_mm(t, st, r1(p['bn_g%d' % i]), r1(p['bn_b%d' % i]),
                          p['conv_w%d' % (i + 1)], dinv)
      else:
        h4 = _tc_norm_last(t, st, r1(p['bn_g%d' % i]), r1(p['bn_b%d' % i]))
    else:
      t = (part[0] + part[1]) * dinv + p['conv_b%d' % i]
      rows = jnp.arange(TBL)[:, None]
      tm = jnp.where(rows < N, t, 0.0)
      mu = jnp.sum(tm, 0) / N
      var = jnp.sum(tm * tm, 0) / N - mu * mu
      hn = (t - mu) * jax.lax.rsqrt(var + 1e-5) * p['bn_g%d' % i] + \
          p['bn_b%d' % i]
      act = jnp.maximum(hn, 0.0) if i == 3 else jnp.where(
          hn > 0, hn, jnp.exp(hn) - 1.0)
      if i < 3:
        hws = (act @ p['conv_w%d' % (i + 1)]) * dinv
      else:
        h4 = act

  if _USE_SC_POOL:
    gs_p, gc_p = _sc_pool(h4, bat_idx)
  else:
    gs_p = jax.ops.segment_sum(h4, bat_pad, num_segments=SEG_R).reshape(
        1, SEG_R, H) * half
    gc_p = jax.ops.segment_sum(
        jnp.ones((TBL, H), jnp.float32), bat_pad, num_segments=SEG_R
    ).reshape(1, SEG_R, H) * half

  if _USE_TC:
    out = _tc_head(gs_p, gc_p, p['head_w1'], r1(p['head_b1']),
                   p['head_w2'], r1(p['head_b2']))
  else:
    gcnt = (gc_p[0] + gc_p[1])[:G, 0:1]
    g = (gs_p[0] + gs_p[1])[:G] / jnp.maximum(gcnt, 1.0)
    pre = g @ p['head_w1'] + p['head_b1']
    z = jnp.where(pre > 0, pre, jnp.exp(pre) - 1.0)
    out = z @ p['head_w2'] + p['head_b2']
  return out


# speedup vs baseline: 14.0599x; 1.0854x over previous
import functools

import jax
import jax.numpy as jnp
from jax import lax
from jax.experimental import pallas as pl
from jax.experimental.pallas import tpu as pltpu
from jax.experimental.pallas import tpu_sc as plsc

N = 10000
E = 320000
D = 128
H = 128
G = 64
S = 64

NC = 2
NS = 16
NW = NC * NS

E_CHUNK = 64
E_BLOCKS = 6
E_CHUNKS_PER_B = 27
E_CHUNKS_PER_W = E_BLOCKS * E_CHUNKS_PER_B
EP = NW * E_CHUNKS_PER_W * E_CHUNK
E_PAD_ROW = N

TBL = 10240
TBL_PER_TILE = TBL // NS

NPT = NW * 3 * 128

P_CHUNK = 64
P_CHUNKS_PER_W = 5

SEG_R = 80
SEG_TRASH = 64

_MESH = plsc.VectorSubcoreMesh(
    core_axis_name="c", subcore_axis_name="s", num_cores=NC, num_subcores=NS
)


def _fill_f32(ref, rows, value):
  cols = ref.shape[1]
  vec = jnp.full((16,), value, dtype=jnp.float32)
  per_row = cols // 16

  def body(i, _):
    r = i // per_row
    c = (i % per_row) * 16
    ref[r, pl.ds(c, 16)] = vec
    return 0

  lax.fori_loop(0, rows * per_row, body, 0)


@functools.partial(
    pl.kernel,
    mesh=_MESH,
    out_type=jax.ShapeDtypeStruct((NC, TBL, H), jnp.float32),
    scratch_types=[
        pltpu.VMEM_SHARED((TBL, H), jnp.float32),
        pltpu.VMEM((E_CHUNKS_PER_B, E_CHUNK), jnp.int32),
        pltpu.VMEM((E_CHUNKS_PER_B, E_CHUNK), jnp.int32),
        pltpu.VMEM((E_CHUNK, H), jnp.float32),
        pltpu.VMEM((E_CHUNK, H), jnp.float32),
        pltpu.VMEM((E_CHUNK, H), jnp.float32),
        pltpu.VMEM((32, H), jnp.float32),
        pltpu.SemaphoreType.DMA,
        pltpu.SemaphoreType.DMA,
        pltpu.SemaphoreType.DMA,
        pltpu.SemaphoreType.DMA,
        pltpu.SemaphoreType.DMA,
        pltpu.SemaphoreType.DMA,
    ],
)
def _sc_message(table, src_hbm, dst_hbm, out, acc, src_v, dst_v, rows0, rows1,
                rows2, zbuf, gsem0, gsem1, gsem2, ssem0, ssem1, ssem2):
  c = lax.axis_index("c")
  s = lax.axis_index("s")
  w = s * NC + c

  _fill_f32(zbuf, 32, 0.0)
  for k in range(TBL_PER_TILE // 32):
    pltpu.sync_copy(zbuf, acc.at[pl.ds(s * TBL_PER_TILE + k * 32, 32)])

  plsc.subcore_barrier()

  bufs = (rows0, rows1, rows2)
  gsems = (gsem0, gsem1, gsem2)
  ssems = (ssem0, ssem1, ssem2)
  for blk in range(E_BLOCKS):
    pltpu.sync_copy(src_hbm.at[w, blk], src_v)
    pltpu.sync_copy(dst_hbm.at[w, blk], dst_v)

    pltpu.async_copy(table.at[src_v.at[0]], rows0, gsem0)
    pltpu.async_copy(table.at[src_v.at[1]], rows1, gsem1)

    def body(j, _):
      for b in range(3):

        @pl.when(j % 3 == b)
        def _(b=b):
          bp = (b + 2) % 3

          @pl.when(j > 0)
          def _():
            pltpu.make_async_copy(
                bufs[bp], acc.at[dst_v.at[j - 1]], ssems[bp]).wait()

          @pl.when(j + 2 < E_CHUNKS_PER_B)
          def _():
            pltpu.async_copy(table.at[src_v.at[j + 2]], bufs[bp], gsems[bp])

          pltpu.make_async_copy(table.at[src_v.at[j]], bufs[b],
                                gsems[b]).wait()
          pltpu.async_copy(bufs[b], acc.at[dst_v.at[j]], ssems[b], add=True)

      return 0

    lax.fori_loop(0, E_CHUNKS_PER_B, body, 0)
    last = E_CHUNKS_PER_B - 1
    pltpu.make_async_copy(bufs[last % 3], acc.at[dst_v.at[last]],
                          ssems[last % 3]).wait()

  plsc.subcore_barrier()

  sl = pl.ds(s * TBL_PER_TILE, TBL_PER_TILE)
  pltpu.sync_copy(acc.at[sl], out.at[c, sl])


@functools.partial(
    pl.kernel,
    mesh=_MESH,
    out_type=(
        jax.ShapeDtypeStruct((NC, TBL, H), jnp.float32),
        jax.ShapeDtypeStruct((NC, SEG_R, D), jnp.float32),
        jax.ShapeDtypeStruct((NC, SEG_R, H), jnp.float32),
    ),
    scratch_types=[
        pltpu.VMEM_SHARED((TBL, H), jnp.float32),
        pltpu.VMEM_SHARED((SEG_R, D), jnp.float32),
        pltpu.VMEM_SHARED((SEG_R, H), jnp.float32),
        pltpu.VMEM((E_CHUNKS_PER_B, E_CHUNK), jnp.int32),
        pltpu.VMEM((3, 128), jnp.int32),
        pltpu.VMEM((128, D), jnp.float32),
        pltpu.VMEM((128, H), jnp.float32),
        pltpu.SemaphoreType.DMA,
    ],
)
def _sc_stats(x_hbm, sid_hbm, dst_hbm, out_deg, out_sum, out_cnt,
              deg_acc, sum_acc, cnt_acc, dst_v, sid_v, xrows, ones, gsem):
  c = lax.axis_index("c")
  s = lax.axis_index("s")
  w = s * NC + c

  _fill_f32(ones, 128, 0.0)
  for k in range(TBL_PER_TILE // 128):
    pltpu.sync_copy(ones, deg_acc.at[pl.ds(s * TBL_PER_TILE + k * 128, 128)])

  @pl.when(s == 0)
  def _():
    pltpu.sync_copy(ones.at[pl.ds(0, SEG_R)], cnt_acc)

  @pl.when(s == 1)
  def _():
    pltpu.sync_copy(ones.at[pl.ds(0, SEG_R)], sum_acc)

  _fill_f32(ones, 128, 1.0)
  pltpu.sync_copy(sid_hbm.at[w], sid_v)

  plsc.subcore_barrier()

  def dbody(j, _):
    pltpu.sync_copy(ones.at[pl.ds(0, E_CHUNK)], deg_acc.at[dst_v.at[j]],
                    add=True)
    return 0

  for blk in range(E_BLOCKS):
    pltpu.sync_copy(dst_hbm.at[w, blk], dst_v)
    lax.fori_loop(0, E_CHUNKS_PER_B, dbody, 0)

  for j in range(3):
    pltpu.async_copy(
        x_hbm.at[pl.ds(w * 384 + j * 128, 128)], xrows, gsem).wait()
    pltpu.sync_copy(xrows, sum_acc.at[sid_v.at[j]], add=True)
    pltpu.sync_copy(ones, cnt_acc.at[sid_v.at[j]], add=True)

  plsc.subcore_barrier()

  sl = pl.ds(s * TBL_PER_TILE, TBL_PER_TILE)
  pltpu.sync_copy(deg_acc.at[sl], out_deg.at[c, sl])

  @pl.when(s == 2)
  def _():
    pltpu.sync_copy(sum_acc, out_sum.at[c])

  @pl.when(s == 3)
  def _():
    pltpu.sync_copy(cnt_acc, out_cnt.at[c])


@functools.partial(
    pl.kernel,
    mesh=_MESH,
    out_type=(
        jax.ShapeDtypeStruct((NC, SEG_R, H), jnp.float32),
        jax.ShapeDtypeStruct((NC, SEG_R, H), jnp.float32),
    ),
    scratch_types=[
        pltpu.VMEM_SHARED((SEG_R, H), jnp.float32),
        pltpu.VMEM_SHARED((SEG_R, H), jnp.float32),
        pltpu.VMEM((P_CHUNKS_PER_W, P_CHUNK), jnp.int32),
        pltpu.VMEM((P_CHUNK, H), jnp.float32),
        pltpu.VMEM((P_CHUNK, H), jnp.float32),
        pltpu.SemaphoreType.DMA,
    ],
)
def _sc_pool(h_hbm, bat_hbm, out_sum, out_cnt, sum_acc, cnt_acc, bat_v, hrows,
             ones, gsem):
  c = lax.axis_index("c")
  s = lax.axis_index("s")
  w = s * NC + c

  _fill_f32(ones, P_CHUNK, 0.0)

  @pl.when(s == 0)
  def _():
    pltpu.sync_copy(ones, sum_acc.at[pl.ds(0, 64)])
    pltpu.sync_copy(ones, sum_acc.at[pl.ds(SEG_R - 64, 64)])

  @pl.when(s == 1)
  def _():
    pltpu.sync_copy(ones, cnt_acc.at[pl.ds(0, 64)])
    pltpu.sync_copy(ones, cnt_acc.at[pl.ds(SEG_R - 64, 64)])

  _fill_f32(ones, P_CHUNK, 1.0)
  pltpu.sync_copy(bat_hbm.at[w], bat_v)

  plsc.subcore_barrier()

  for j in range(P_CHUNKS_PER_W):
    pltpu.async_copy(
        h_hbm.at[pl.ds(w * P_CHUNKS_PER_W * P_CHUNK + j * P_CHUNK, P_CHUNK)],
        hrows, gsem).wait()
    pltpu.sync_copy(hrows, sum_acc.at[bat_v.at[j]], add=True)
    pltpu.sync_copy(ones, cnt_acc.at[bat_v.at[j]], add=True)

  plsc.subcore_barrier()

  @pl.when(s == 0)
  def _():
    pltpu.sync_copy(sum_acc, out_sum.at[c])

  @pl.when(s == 1)
  def _():
    pltpu.sync_copy(cnt_acc, out_cnt.at[c])


def _elu(x):
  return jnp.where(x > 0, x, jnp.exp(x) - 1.0)


def _mm(a, b):
  return jax.lax.dot_general(a, b, (((1,), (0,)), ((), ())),
                             preferred_element_type=jnp.float32)


def _tc_dinv_body(deg_ref, out_ref):
  deg = deg_ref[0, :, 0:1] + deg_ref[1, :, 0:1]
  out_ref[...] = jnp.where(deg > 0, jax.lax.rsqrt(deg), 0.0)


def _tc_dinv(deg_p):
  blk = 1024
  return pl.pallas_call(
      _tc_dinv_body,
      grid=(TBL // blk,),
      in_specs=[pl.BlockSpec((NC, blk, H), lambda b: (0, b, 0))],
      out_specs=pl.BlockSpec((blk, 1), lambda b: (b, 0)),
      out_shape=jax.ShapeDtypeStruct((TBL, 1), jnp.float32),
  )(deg_p)


def _tc_attn_body(sum_ref, cnt_ref, w1_ref, b1_ref, w2_ref, out_ref):
  cnt = cnt_ref[0, :S, 0:1] + cnt_ref[1, :S, 0:1]
  sums = sum_ref[0, :S, :] + sum_ref[1, :S, :]
  sm = sums / jnp.maximum(cnt, 1.0)
  hdn = jnp.tanh(_mm(sm, w1_ref[...]) + b1_ref[...])
  scores = _mm(hdn, w2_ref[...])
  scores = jnp.where(cnt > 0, scores, -1e30)
  m = jnp.max(scores)
  e = jnp.exp(scores - m)
  out_ref[...] = e / jnp.sum(e)


def _tc_attn(sums_p, cnt_p, w1, b1, w2):
  return pl.pallas_call(
      _tc_attn_body,
      out_shape=jax.ShapeDtypeStruct((S, 1), jnp.float32),
  )(sums_p, cnt_p, w1, b1, w2)


def _tc_prep_body(x_ref, dinv_ref, attn_ref, w0a_ref, w0b_ref, out_ref):
  x = x_ref[...]
  sid = x[:, 5:6].astype(jnp.int32)
  onehot = (sid == lax.broadcasted_iota(jnp.int32, (1, S), 1)
            ).astype(jnp.float32)
  attn = _mm(onehot, attn_ref[...])
  hw = _mm(x, w0a_ref[...]) + attn * w0b_ref[...]
  out_ref[...] = hw * dinv_ref[...]


def _tc_prep(x_pad, dinv, attn, w0a, w0b):
  blk = 1024
  return pl.pallas_call(
      _tc_prep_body,
      grid=(TBL // blk,),
      in_specs=[
          pl.BlockSpec((blk, D), lambda b: (b, 0)),
          pl.BlockSpec((blk, 1), lambda b: (b, 0)),
          pl.BlockSpec((S, 1), lambda b: (0, 0)),
          pl.BlockSpec((D, H), lambda b: (0, 0)),
          pl.BlockSpec((1, H), lambda b: (0, 0)),
      ],
      out_specs=pl.BlockSpec((blk, H), lambda b: (b, 0)),
      out_shape=jax.ShapeDtypeStruct((TBL, H), jnp.float32),
  )(x_pad, dinv, attn, w0a, w0b)


def _tc_combine_body(p_ref, dinv_ref, b_ref, t_ref, st_ref):
  blk = t_ref.shape[0]
  b = pl.program_id(0)
  t = (p_ref[0] + p_ref[1]) * dinv_ref[...] + b_ref[...]
  t_ref[...] = t
  rows = b * blk + lax.broadcasted_iota(jnp.int32, (blk, 1), 0)
  tm = jnp.where(rows < N, t, 0.0)

  @pl.when(b == 0)
  def _():
    st_ref[...] = jnp.zeros_like(st_ref)

  st_ref[0:1, :] += jnp.sum(tm, axis=0, keepdims=True)
  st_ref[1:2, :] += jnp.sum(tm * tm, axis=0, keepdims=True)


def _tc_combine(p, dinv, bias):
  blk = 1024
  return pl.pallas_call(
      _tc_combine_body,
      grid=(TBL // blk,),
      in_specs=[
          pl.BlockSpec((NC, blk, H), lambda b: (0, b, 0)),
          pl.BlockSpec((blk, 1), lambda b: (b, 0)),
          pl.BlockSpec((1, H), lambda b: (0, 0)),
      ],
      out_specs=[
          pl.BlockSpec((blk, H), lambda b: (b, 0)),
          pl.BlockSpec((8, H), lambda b: (0, 0)),
      ],
      out_shape=[
          jax.ShapeDtypeStruct((TBL, H), jnp.float32),
          jax.ShapeDtypeStruct((8, H), jnp.float32),
      ],
  )(p, dinv, bias)


def _bn_act(t_ref, st_ref, g_ref, bb_ref, last):
  mu = st_ref[0:1, :] * (1.0 / N)
  ex2 = st_ref[1:2, :] * (1.0 / N)
  var = ex2 - mu * mu
  inv = jax.lax.rsqrt(var + 1e-5)
  hn = (t_ref[...] - mu) * (inv * g_ref[...]) + bb_ref[...]
  return jnp.maximum(hn, 0.0) if last else _elu(hn)


def _tc_norm_mm_body(t_ref, st_ref, g_ref, bb_ref, w_ref, dinv_ref, out_ref):
  h = _bn_act(t_ref, st_ref, g_ref, bb_ref, last=False)
  out_ref[...] = _mm(h, w_ref[...]) * dinv_ref[...]


def _tc_norm_mm(t, st, g, bb, w, dinv):
  blk = 1024
  return pl.pallas_call(
      _tc_norm_mm_body,
      grid=(TBL // blk,),
      in_specs=[
          pl.BlockSpec((blk, H), lambda b: (b, 0)),
          pl.BlockSpec((8, H), lambda b: (0, 0)),
          pl.BlockSpec((1, H), lambda b: (0, 0)),
          pl.BlockSpec((1, H), lambda b: (0, 0)),
          pl.BlockSpec((H, H), lambda b: (0, 0)),
          pl.BlockSpec((blk, 1), lambda b: (b, 0)),
      ],
      out_specs=pl.BlockSpec((blk, H), lambda b: (b, 0)),
      out_shape=jax.ShapeDtypeStruct((TBL, H), jnp.float32),
  )(t, st, g, bb, w, dinv)


def _tc_norm_last_body(t_ref, st_ref, g_ref, bb_ref, out_ref):
  out_ref[...] = _bn_act(t_ref, st_ref, g_ref, bb_ref, last=True)


def _tc_norm_last(t, st, g, bb):
  blk = 1024
  return pl.pallas_call(
      _tc_norm_last_body,
      grid=(TBL // blk,),
      in_specs=[
          pl.BlockSpec((blk, H), lambda b: (b, 0)),
          pl.BlockSpec((8, H), lambda b: (0, 0)),
          pl.BlockSpec((1, H), lambda b: (0, 0)),
          pl.BlockSpec((1, H), lambda b: (0, 0)),
      ],
      out_specs=pl.BlockSpec((blk, H), lambda b: (b, 0)),
      out_shape=jax.ShapeDtypeStruct((TBL, H), jnp.float32),
  )(t, st, g, bb)


def _tc_head_body(gs_ref, gc_ref, w1_ref, b1_ref, w2_ref, b2_ref, out_ref):
  cnt = gc_ref[0, :G, 0:1] + gc_ref[1, :G, 0:1]
  gsum = gs_ref[0, :G, :] + gs_ref[1, :G, :]
  g = gsum / jnp.maximum(cnt, 1.0)
  z = _elu(_mm(g, w1_ref[...]) + b1_ref[...])
  out_ref[...] = _mm(z, w2_ref[...]) + b2_ref[...]


def _tc_head(gs_p, gc_p, w1, b1, w2, b2):
  return pl.pallas_call(
      _tc_head_body,
      out_shape=jax.ShapeDtypeStruct((G, 1), jnp.float32),
  )(gs_p, gc_p, w1, b1, w2, b2)


_USE_SC_STATS = True
_USE_SC_MSG = True
_USE_SC_POOL = True
_USE_TC = True


def kernel(x, params, edge_index, batch):
  sid = x[:, 5].astype(jnp.int32)
  sid_pad = jnp.full((NPT,), SEG_TRASH, jnp.int32).at[:N].set(sid)
  sid_idx = sid_pad.reshape(NW, 3, 128)

  x_pad = jnp.zeros((NPT, D), jnp.float32).at[:N].set(x)

  loop = jnp.arange(N, dtype=jnp.int32)
  epad = jnp.full((EP - E - N,), E_PAD_ROW, jnp.int32)
  src = jnp.concatenate([edge_index[0].astype(jnp.int32), loop, epad])
  dst = jnp.concatenate([edge_index[1].astype(jnp.int32), loop, epad])
  src_idx = src.reshape(NW, E_BLOCKS, E_CHUNKS_PER_B, E_CHUNK)
  dst_idx = dst.reshape(NW, E_BLOCKS, E_CHUNKS_PER_B, E_CHUNK)

  bat_pad = jnp.full((TBL,), SEG_TRASH, jnp.int32).at[:N].set(
      batch.astype(jnp.int32))
  bat_idx = bat_pad.reshape(NW, P_CHUNKS_PER_W, P_CHUNK)

  p = params
  r1 = lambda a: a.reshape(1, -1)
  half = jnp.array([0.5, 0.5]).reshape(NC, 1, 1)

  if _USE_SC_STATS:
    deg_p, sum_p, cnt_p = _sc_stats(x_pad, sid_idx, dst_idx)
  else:
    deg_p = jax.ops.segment_sum(
        jnp.ones((EP, H), jnp.float32), dst, num_segments=TBL
    ).reshape(1, TBL, H) * half
    sum_p = jax.ops.segment_sum(x, sid, num_segments=SEG_R).reshape(
        1, SEG_R, D) * half
    cnt_p = jax.ops.segment_sum(
        jnp.ones((N, H), jnp.float32), sid, num_segments=SEG_R
    ).reshape(1, SEG_R, H) * half

  if _USE_TC:
    dinv = _tc_dinv(deg_p)
    attn = _tc_attn(sum_p, cnt_p, p['attn_w1'], r1(p['attn_b1']),
                    p['attn_w2'])
    w0 = p['conv_w0']
    hws = _tc_prep(x_pad[:TBL], dinv, attn, w0[:D], w0[D:D + 1])
  else:
    cnt = (cnt_p[0] + cnt_p[1])[:S, 0:1]
    sums = (sum_p[0] + sum_p[1])[:S]
    sm = sums / jnp.maximum(cnt, 1.0)
    hdn = jnp.tanh(sm @ p['attn_w1'] + p['attn_b1'])
    scores = jnp.where(cnt > 0, hdn @ p['attn_w2'], -1e30)
    e = jnp.exp(scores - jnp.max(scores))
    attn = e / jnp.sum(e)
    deg = (deg_p[0] + deg_p[1])[:, 0:1]
    dinv = jnp.where(deg > 0, jax.lax.rsqrt(deg), 0.0)
    sid_t = jnp.zeros((TBL,), jnp.int32).at[:N].set(sid)
    av = attn[sid_t]
    w0 = p['conv_w0']
    hws = (x_pad[:TBL] @ w0[:D] + av * w0[D:D + 1]) * dinv

  for i in range(4):
    if _USE_SC_MSG:
      part = _sc_message(hws, src_idx, dst_idx)
    else:
      part = jax.ops.segment_sum(hws[src], dst, num_segments=TBL).reshape(
          1, TBL, H) * half
    if _USE_TC:
      t, st = _tc_combine(part, dinv, r1(p['conv_b%d' % i]))
      if i < 3:
        hws = _tc_norm_mm(t, st, r1(p['bn_g%d' % i]), r1(p['bn_b%d' % i]),
                          p['conv_w%d' % (i + 1)], dinv)
      else:
        h4 = _tc_norm_last(t, st, r1(p['bn_g%d' % i]), r1(p['bn_b%d' % i]))
    else:
      t = (part[0] + part[1]) * dinv + p['conv_b%d' % i]
      rows = jnp.arange(TBL)[:, None]
      tm = jnp.where(rows < N, t, 0.0)
      mu = jnp.sum(tm, 0) / N
      var = jnp.sum(tm * tm, 0) / N - mu * mu
      hn = (t - mu) * jax.lax.rsqrt(var + 1e-5) * p['bn_g%d' % i] + \
          p['bn_b%d' % i]
      act = jnp.maximum(hn, 0.0) if i == 3 else jnp.where(
          hn > 0, hn, jnp.exp(hn) - 1.0)
      if i < 3:
        hws = (act @ p['conv_w%d' % (i + 1)]) * dinv
      else:
        h4 = act

  if _USE_SC_POOL:
    gs_p, gc_p = _sc_pool(h4, bat_idx)
  else:
    gs_p = jax.ops.segment_sum(h4, bat_pad, num_segments=SEG_R).reshape(
        1, SEG_R, H) * half
    gc_p = jax.ops.segment_sum(
        jnp.ones((TBL, H), jnp.float32), bat_pad, num_segments=SEG_R
    ).reshape(1, SEG_R, H) * half

  if _USE_TC:
    out = _tc_head(gs_p, gc_p, p['head_w1'], r1(p['head_b1']),
                   p['head_w2'], r1(p['head_b2']))
  else:
    gcnt = (gc_p[0] + gc_p[1])[:G, 0:1]
    g = (gs_p[0] + gs_p[1])[:G] / jnp.maximum(gcnt, 1.0)
    pre = g @ p['head_w1'] + p['head_b1']
    z = jnp.where(pre > 0, pre, jnp.exp(pre) - 1.0)
    out = z @ p['head_w2'] + p['head_b2']
  return out
